# baseline (speedup 1.0000x reference)
import functools

import jax
import jax.numpy as jnp
from jax import lax
from jax.experimental import pallas as pl
from jax.experimental.pallas import tpu as pltpu
from jax.experimental.pallas import tpu_sc as plsc

_N = 10000
_E = 320000
_G = 32
_D = 128
_DW = 144
_H = 4
_NA = 10112
_TILES = 32
_K = 112
_CHUNKS = 90
_ET = _CHUNKS * _K
_EP = _TILES * _ET
_BN = 632
_NB = _NA // _BN
_BN3 = 128
_NB3 = _NA // _BN3
_GU = 4
_SS = 9
_NS = _CHUNKS // _SS
_HI = lax.Precision.HIGHEST


def _tc_feat_body(x_ref, w_ref, as_ref, ad_ref, th_ref, tad_ref):
    h = jnp.dot(x_ref[...], w_ref[...], preferred_element_type=jnp.float32,
                precision=_HI)
    th_ref[:, pl.ds(0, _D)] = h
    th_ref[:, pl.ds(_D, 16)] = jnp.dot(h, as_ref[...],
                                       preferred_element_type=jnp.float32,
                                       precision=_HI)
    tad_ref[...] = jnp.dot(h, ad_ref[...], preferred_element_type=jnp.float32,
                           precision=_HI)


def _tc1(x, w1, as1m, ad1m):
    return pl.pallas_call(
        _tc_feat_body,
        grid=(25,),
        in_specs=[
            pl.BlockSpec((400, 159), lambda i: (i, 0)),
            pl.BlockSpec((159, _D), lambda i: (0, 0)),
            pl.BlockSpec((_D, 16), lambda i: (0, 0)),
            pl.BlockSpec((_D, 16), lambda i: (0, 0)),
        ],
        out_specs=[
            pl.BlockSpec((400, _DW), lambda i: (i, 0)),
            pl.BlockSpec((400, 16), lambda i: (i, 0)),
        ],
        out_shape=[
            jax.ShapeDtypeStruct((_NA, _DW), jnp.float32),
            jax.ShapeDtypeStruct((_NA, 16), jnp.float32),
        ],
    )(x, w1, as1m, ad1m)


def _tc2_body(acc_ref, b1_ref, w2_ref, as_ref, ad_ref, r_ref,
              th_ref, tad_ref):
    i = pl.program_id(0)
    den = acc_ref[0, :, pl.ds(_D, 16)] + acc_ref[1, :, pl.ds(_D, 16)]
    rden = 1.0 / (den + 1e-16)
    denx = jnp.dot(rden, r_ref[...], preferred_element_type=jnp.float32,
                   precision=_HI)
    hsum = acc_ref[0, :, pl.ds(0, _D)] + acc_ref[1, :, pl.ds(0, _D)]
    g = hsum * denx + b1_ref[...]
    g = jnp.where(g > 0, g, jnp.exp(g) - 1.0)
    rows = lax.broadcasted_iota(jnp.int32, (_BN, 1), 0) + i * _BN
    g = jnp.where(rows < _N, g, 0.0)
    h2 = jnp.dot(g, w2_ref[...], preferred_element_type=jnp.float32,
                 precision=_HI)
    th_ref[:, pl.ds(0, _D)] = h2
    th_ref[:, pl.ds(_D, 16)] = jnp.dot(h2, as_ref[...],
                                       preferred_element_type=jnp.float32,
                                       precision=_HI)
    tad_ref[...] = jnp.dot(h2, ad_ref[...], preferred_element_type=jnp.float32,
                           precision=_HI)


def _tc2(acc, b1r, w2, as2m, ad2m, rmat):
    return pl.pallas_call(
        _tc2_body,
        grid=(_NB,),
        in_specs=[
            pl.BlockSpec((2, _BN, _DW), lambda i: (0, i, 0)),
            pl.BlockSpec((1, _D), lambda i: (0, 0)),
            pl.BlockSpec((_D, _D), lambda i: (0, 0)),
            pl.BlockSpec((_D, 16), lambda i: (0, 0)),
            pl.BlockSpec((_D, 16), lambda i: (0, 0)),
            pl.BlockSpec((16, _D), lambda i: (0, 0)),
        ],
        out_specs=[
            pl.BlockSpec((_BN, _DW), lambda i: (i, 0)),
            pl.BlockSpec((_BN, 16), lambda i: (i, 0)),
        ],
        out_shape=[
            jax.ShapeDtypeStruct((_NA, _DW), jnp.float32),
            jax.ShapeDtypeStruct((_NA, 16), jnp.float32),
        ],
    )(acc, b1r, w2, as2m, ad2m, rmat)


def _tc3_body(acc_ref, b2_ref, r_ref, bat_ref, wlin_ref, blin_ref,
              out_ref, sum_ref, cnt_ref, mx_ref):
    i = pl.program_id(0)
    den = acc_ref[0, :, pl.ds(_D, 16)] + acc_ref[1, :, pl.ds(_D, 16)]
    rden = 1.0 / (den + 1e-16)
    denx = jnp.dot(rden, r_ref[...], preferred_element_type=jnp.float32,
                   precision=_HI)
    hsum = acc_ref[0, :, pl.ds(0, _D)] + acc_ref[1, :, pl.ds(0, _D)]
    h2 = hsum * denx + b2_ref[...]
    bids = bat_ref[...]
    gio = lax.broadcasted_iota(jnp.int32, (_BN3, _G), 1)
    onehot = (bids == gio).astype(jnp.float32)

    @pl.when(i == 0)
    def _init():
        sum_ref[...] = jnp.zeros((_G, _D), jnp.float32)
        cnt_ref[...] = jnp.zeros((_G, _D), jnp.float32)
        mx_ref[...] = jnp.full((_G, _D), -1e30, jnp.float32)

    dn = (((0,), (0,)), ((), ()))
    sum_ref[...] += lax.dot_general(onehot, h2, dn,
                                    preferred_element_type=jnp.float32,
                                    precision=_HI)
    cnt_ref[...] += lax.dot_general(onehot, jnp.ones((_BN3, _D), jnp.float32),
                                    dn, preferred_element_type=jnp.float32,
                                    precision=_HI)
    masked = jnp.where(onehot[:, :, None] > 0, h2[:, None, :], -1e30)
    mx_ref[...] = jnp.maximum(mx_ref[...], jnp.max(masked, axis=0))

    @pl.when(i == _NB3 - 1)
    def _fin():
        cnt = cnt_ref[...]
        pooled = sum_ref[...] / jnp.maximum(cnt, 1.0)
        pooled = pooled + jnp.where(cnt > 0, mx_ref[...], 0.0)
        pred = lax.dot_general(wlin_ref[...], pooled, (((1,), (1,)), ((), ())),
                               preferred_element_type=jnp.float32,
                               precision=_HI)
        out_ref[...] = pred + blin_ref[...]


def _tc3(acc, b2r, rmat, bat2d, wlint, blin2d):
    return pl.pallas_call(
        _tc3_body,
        grid=(_NB3,),
        in_specs=[
            pl.BlockSpec((2, _BN3, _DW), lambda i: (0, i, 0)),
            pl.BlockSpec((1, _D), lambda i: (0, 0)),
            pl.BlockSpec((16, _D), lambda i: (0, 0)),
            pl.BlockSpec((_BN3, 1), lambda i: (i, 0)),
            pl.BlockSpec((1, _D), lambda i: (0, 0)),
            pl.BlockSpec((1, 1), lambda i: (0, 0)),
        ],
        out_specs=pl.BlockSpec((1, _G), lambda i: (0, 0)),
        out_shape=jax.ShapeDtypeStruct((1, _G), jnp.float32),
        scratch_shapes=[
            pltpu.VMEM((_G, _D), jnp.float32),
            pltpu.VMEM((_G, _D), jnp.float32),
            pltpu.VMEM((_G, _D), jnp.float32),
        ],
    )(acc, b2r, rmat, bat2d, wlint, blin2d)


def _splat(vec, k):
    dnums = lax.GatherDimensionNumbers(
        offset_dims=(), collapsed_slice_dims=(0,), start_index_map=(0,))
    idx = jnp.full((16, 1), k, jnp.int32)
    return lax.gather(vec, idx, dnums, (1,),
                      mode=lax.GatherScatterMode.PROMISE_IN_BOUNDS)


_sc_mesh = plsc.VectorSubcoreMesh(
    core_axis_name="c", subcore_axis_name="s", num_cores=2, num_subcores=16)


@functools.partial(
    pl.kernel,
    out_type=jax.ShapeDtypeStruct((2, _NA, _DW), jnp.float32),
    mesh=_sc_mesh,
    scratch_types=[
        pltpu.VMEM_SHARED((_NA, _DW), jnp.float32),
        pltpu.VMEM((_SS, _K), jnp.int32),
        pltpu.VMEM((_SS, _K), jnp.int32),
        pltpu.VMEM((_SS, _K), jnp.int32),
        pltpu.VMEM((_SS, _K), jnp.int32),
        pltpu.VMEM((_K, _DW), jnp.float32),
        pltpu.VMEM((_K, _DW), jnp.float32),
        pltpu.VMEM((_K, 16), jnp.float32),
        pltpu.VMEM((_K, 16), jnp.float32),
        pltpu.SemaphoreType.DMA,
        pltpu.SemaphoreType.DMA,
        pltpu.SemaphoreType.DMA,
        pltpu.SemaphoreType.DMA,
        pltpu.SemaphoreType.DMA,
        pltpu.SemaphoreType.DMA,
    ],
    compiler_params=pltpu.CompilerParams(
        needs_layout_passes=False, use_tc_tiling_on_sc=False),
)
def _sc_layer(th, tad, srcp, dstp, acc_out,
              acc, sia0, sia1, dia0, dia1, hbuf0, hbuf1, adbuf0, adbuf1,
              gh0, gh1, ga0, ga1, sc0, sc1):
    cid = lax.axis_index("c")
    sid = lax.axis_index("s")
    wid = sid * 2 + cid
    zed = jnp.zeros((16,), jnp.float32)
    sias = (sia0, sia1)
    dias = (dia0, dia1)
    hbufs = (hbuf0, hbuf1)
    adbufs = (adbuf0, adbuf1)
    ghs = (gh0, gh1)
    gas = (ga0, ga1)
    scs = (sc0, sc1)

    def _zrow(r, carry):
        for j in range(_DW // 16):
            hbuf0[r, pl.ds(j * 16, 16)] = zed
        return carry

    lax.fori_loop(0, _K, _zrow, 0)
    rows_per_sub = _NA // 16
    nfull = rows_per_sub // _K
    rem = rows_per_sub % _K
    for z in range(nfull):
        base = sid * rows_per_sub + z * _K
        pltpu.sync_copy(hbuf0, acc.at[pl.ds(base, _K), :])
    if rem:
        pltpu.sync_copy(hbuf0.at[pl.ds(0, rem), :],
                        acc.at[pl.ds(sid * rows_per_sub + nfull * _K, rem), :])
    plsc.subcore_barrier()

    lanes = lax.iota(jnp.int32, 16)
    sub = lanes >> 2
    headv = lanes & 3

    def _idx_fetch(s, ib):
        row0 = wid * _CHUNKS + s * _SS
        pltpu.sync_copy(srcp.at[pl.ds(row0, _SS), :], sias[ib])
        pltpu.sync_copy(dstp.at[pl.ds(row0, _SS), :], dias[ib])

    def _gath(ib, k, hb):
        pltpu.async_copy(th.at[sias[ib].at[k]], hbufs[hb], ghs[hb])
        pltpu.async_copy(tad.at[dias[ib].at[k]], adbufs[hb], gas[hb])

    def _compute(b):
        hb = hbufs[b]
        ab = adbufs[b]

        def _groups(gi, carry2):
            for gu in range(_GU):
                g = gi * _GU + gu
                rowv = sub + 4 * g
                asv = plsc.load_gather(hb, [rowv, _D + headv])
                adv = plsc.load_gather(ab, [rowv, headv])
                e = asv + adv
                e = jnp.where(e > 0, e, 0.2 * e)
                ex = jnp.exp(e)
                plsc.store_scatter(hb, [rowv, _D + headv], ex)
                for j in range(4):
                    r = 4 * g + j
                    for hh in range(_H):
                        spl = _splat(ex, 4 * j + hh)
                        for q in range(2):
                            c0 = hh * 32 + q * 16
                            hb[r, pl.ds(c0, 16)] = hb[r, pl.ds(c0, 16)] * spl
            return carry2

        lax.fori_loop(0, (_K // 4) // _GU, _groups, 0)

    _idx_fetch(0, 0)
    _gath(0, 0, 0)

    def _pair(i2, carry):
        for si in range(2):
            for k in range(_SS):
                s = i2 * 2 + si
                ci = s * _SS + k
                b = (si * _SS + k) & 1
                nb = 1 - b

                if k == 0:
                    @pl.when(s + 1 < _NS)
                    def _pfidx():
                        @pl.when(ci >= 1)
                        def _drain0():
                            pltpu.make_async_copy(
                                hbufs[nb], acc.at[dias[si].at[0]],
                                scs[nb]).wait()
                        _idx_fetch(s + 1, 1 - si)

                @pl.when(ci + 1 < _CHUNKS)
                def _pref():
                    if k == 0:
                        cond = (ci >= 1) & (s + 1 >= _NS)
                    else:
                        cond = ci >= 1

                    @pl.when(cond)
                    def _drain():
                        pltpu.make_async_copy(
                            hbufs[nb], acc.at[dias[si].at[0]], scs[nb]).wait()
                    nk = k + 1
                    if nk < _SS:
                        _gath(si, nk, nb)
                    else:
                        _gath(1 - si, 0, nb)

                pltpu.make_async_copy(th.at[sias[si].at[k]],
                                      hbufs[b], ghs[b]).wait()
                pltpu.make_async_copy(tad.at[dias[si].at[k]],
                                      adbufs[b], gas[b]).wait()
                _compute(b)
                pltpu.async_copy(hbufs[b], acc.at[dias[si].at[k]],
                                 scs[b], add=True)
        return carry

    lax.fori_loop(0, _NS // 2, _pair, 0)
    pltpu.make_async_copy(hbufs[0], acc.at[dias[1].at[0]], scs[0]).wait()
    pltpu.make_async_copy(hbufs[1], acc.at[dias[1].at[0]], scs[1]).wait()
    plsc.subcore_barrier()
    base = sid * rows_per_sub
    pltpu.sync_copy(acc.at[pl.ds(base, rows_per_sub), :],
                    acc_out.at[cid, pl.ds(base, rows_per_sub), :])


def _att_mat(att):
    cols = jnp.repeat(jnp.arange(_H), 32)
    m = jnp.zeros((_D, 16), jnp.float32)
    return m.at[jnp.arange(_D), cols].set(att.reshape(-1))


def kernel(x, edge_index, batch, W1, att_src1, att_dst1, b1,
           W2, att_src2, att_dst2, b2, W_lin, b_lin):
    f32 = jnp.float32
    as1m = _att_mat(att_src1)
    ad1m = _att_mat(att_dst1)
    as2m = _att_mat(att_src2)
    ad2m = _att_mat(att_dst2)
    rmat = (jnp.arange(16)[:, None] == (jnp.arange(_D) // 32)[None, :]).astype(f32)
    b1r = b1.reshape(1, _D).astype(f32)
    b2r = b2.reshape(1, _D).astype(f32)
    wlint = W_lin.reshape(1, _D).astype(f32)
    blin2d = b_lin.reshape(1, 1).astype(f32)
    src = edge_index[0].astype(jnp.int32)
    dst = edge_index[1].astype(jnp.int32)
    srcp = jnp.zeros((_EP,), jnp.int32).at[:_E].set(src).reshape(_EP // _K, _K)
    dstp = jnp.full((_EP,), _N, jnp.int32).at[:_E].set(dst).reshape(_EP // _K, _K)
    bat2d = jnp.full((_NA, 1), _G, jnp.int32).at[:_N, 0].set(batch.astype(jnp.int32))

    th1, tad1 = _tc1(x.astype(f32), W1.astype(f32), as1m, ad1m)
    acc1 = _sc_layer(th1, tad1, srcp, dstp)
    th2, tad2 = _tc2(acc1, b1r, W2.astype(f32), as2m, ad2m, rmat)
    acc2 = _sc_layer(th2, tad2, srcp, dstp)
    return _tc3(acc2, b2r, rmat, bat2d, wlint, blin2d)


# speedup vs baseline: 1.0200x; 1.0200x over previous
import functools

import jax
import jax.numpy as jnp
from jax import lax
from jax.experimental import pallas as pl
from jax.experimental.pallas import tpu as pltpu
from jax.experimental.pallas import tpu_sc as plsc

_N = 10000
_E = 320000
_G = 32
_D = 128
_DW = 144
_H = 4
_NA = 10112
_TILES = 32
_K = 112
_CHUNKS = 90
_ET = _CHUNKS * _K
_EP = _TILES * _ET
_BN = 632
_NB = _NA // _BN
_BN3 = 128
_NB3 = _NA // _BN3
_GU = 4
_SS = 5
_NS = _CHUNKS // _SS
_HI = lax.Precision.HIGHEST


def _tc_feat_body(x_ref, w_ref, as_ref, ad_ref, th_ref, tad_ref):
    h = jnp.dot(x_ref[...], w_ref[...], preferred_element_type=jnp.float32,
                precision=_HI)
    th_ref[:, pl.ds(0, _D)] = h
    th_ref[:, pl.ds(_D, 16)] = jnp.dot(h, as_ref[...],
                                       preferred_element_type=jnp.float32,
                                       precision=_HI)
    tad_ref[...] = jnp.dot(h, ad_ref[...], preferred_element_type=jnp.float32,
                           precision=_HI)


def _tc1(x_pad, w1p, as1m, ad1m):
    return pl.pallas_call(
        _tc_feat_body,
        grid=(_NB,),
        in_specs=[
            pl.BlockSpec((_BN, 160), lambda i: (i, 0)),
            pl.BlockSpec((160, _D), lambda i: (0, 0)),
            pl.BlockSpec((_D, 16), lambda i: (0, 0)),
            pl.BlockSpec((_D, 16), lambda i: (0, 0)),
        ],
        out_specs=[
            pl.BlockSpec((_BN, _DW), lambda i: (i, 0)),
            pl.BlockSpec((_BN, 16), lambda i: (i, 0)),
        ],
        out_shape=[
            jax.ShapeDtypeStruct((_NA, _DW), jnp.float32),
            jax.ShapeDtypeStruct((_NA, 16), jnp.float32),
        ],
    )(x_pad, w1p, as1m, ad1m)


def _tc2_body(acc_ref, b1_ref, w2_ref, as_ref, ad_ref, r_ref,
              th_ref, tad_ref):
    i = pl.program_id(0)
    den = acc_ref[0, :, pl.ds(_D, 16)] + acc_ref[1, :, pl.ds(_D, 16)]
    rden = 1.0 / (den + 1e-16)
    denx = jnp.dot(rden, r_ref[...], preferred_element_type=jnp.float32,
                   precision=_HI)
    hsum = acc_ref[0, :, pl.ds(0, _D)] + acc_ref[1, :, pl.ds(0, _D)]
    g = hsum * denx + b1_ref[...]
    g = jnp.where(g > 0, g, jnp.exp(g) - 1.0)
    rows = lax.broadcasted_iota(jnp.int32, (_BN, 1), 0) + i * _BN
    g = jnp.where(rows < _N, g, 0.0)
    h2 = jnp.dot(g, w2_ref[...], preferred_element_type=jnp.float32,
                 precision=_HI)
    th_ref[:, pl.ds(0, _D)] = h2
    th_ref[:, pl.ds(_D, 16)] = jnp.dot(h2, as_ref[...],
                                       preferred_element_type=jnp.float32,
                                       precision=_HI)
    tad_ref[...] = jnp.dot(h2, ad_ref[...], preferred_element_type=jnp.float32,
                           precision=_HI)


def _tc2(acc, b1r, w2, as2m, ad2m, rmat):
    return pl.pallas_call(
        _tc2_body,
        grid=(_NB,),
        in_specs=[
            pl.BlockSpec((2, _BN, _DW), lambda i: (0, i, 0)),
            pl.BlockSpec((1, _D), lambda i: (0, 0)),
            pl.BlockSpec((_D, _D), lambda i: (0, 0)),
            pl.BlockSpec((_D, 16), lambda i: (0, 0)),
            pl.BlockSpec((_D, 16), lambda i: (0, 0)),
            pl.BlockSpec((16, _D), lambda i: (0, 0)),
        ],
        out_specs=[
            pl.BlockSpec((_BN, _DW), lambda i: (i, 0)),
            pl.BlockSpec((_BN, 16), lambda i: (i, 0)),
        ],
        out_shape=[
            jax.ShapeDtypeStruct((_NA, _DW), jnp.float32),
            jax.ShapeDtypeStruct((_NA, 16), jnp.float32),
        ],
    )(acc, b1r, w2, as2m, ad2m, rmat)


def _tc3_body(acc_ref, b2_ref, r_ref, bat_ref, wlin_ref, blin_ref,
              out_ref, sum_ref, cnt_ref, mx_ref):
    i = pl.program_id(0)
    den = acc_ref[0, :, pl.ds(_D, 16)] + acc_ref[1, :, pl.ds(_D, 16)]
    rden = 1.0 / (den + 1e-16)
    denx = jnp.dot(rden, r_ref[...], preferred_element_type=jnp.float32,
                   precision=_HI)
    hsum = acc_ref[0, :, pl.ds(0, _D)] + acc_ref[1, :, pl.ds(0, _D)]
    h2 = hsum * denx + b2_ref[...]
    bids = bat_ref[...]
    gio = lax.broadcasted_iota(jnp.int32, (_BN3, _G), 1)
    onehot = (bids == gio).astype(jnp.float32)

    @pl.when(i == 0)
    def _init():
        sum_ref[...] = jnp.zeros((_G, _D), jnp.float32)
        cnt_ref[...] = jnp.zeros((_G, _D), jnp.float32)
        mx_ref[...] = jnp.full((_G, _D), -1e30, jnp.float32)

    dn = (((0,), (0,)), ((), ()))
    sum_ref[...] += lax.dot_general(onehot, h2, dn,
                                    preferred_element_type=jnp.float32,
                                    precision=_HI)
    cnt_ref[...] += lax.dot_general(onehot, jnp.ones((_BN3, _D), jnp.float32),
                                    dn, preferred_element_type=jnp.float32,
                                    precision=_HI)
    masked = jnp.where(onehot[:, :, None] > 0, h2[:, None, :], -1e30)
    mx_ref[...] = jnp.maximum(mx_ref[...], jnp.max(masked, axis=0))

    @pl.when(i == _NB3 - 1)
    def _fin():
        cnt = cnt_ref[...]
        pooled = sum_ref[...] / jnp.maximum(cnt, 1.0)
        pooled = pooled + jnp.where(cnt > 0, mx_ref[...], 0.0)
        pred = lax.dot_general(wlin_ref[...], pooled, (((1,), (1,)), ((), ())),
                               preferred_element_type=jnp.float32,
                               precision=_HI)
        out_ref[...] = pred + blin_ref[...]


def _tc3(acc, b2r, rmat, bat2d, wlint, blin2d):
    return pl.pallas_call(
        _tc3_body,
        grid=(_NB3,),
        in_specs=[
            pl.BlockSpec((2, _BN3, _DW), lambda i: (0, i, 0)),
            pl.BlockSpec((1, _D), lambda i: (0, 0)),
            pl.BlockSpec((16, _D), lambda i: (0, 0)),
            pl.BlockSpec((_BN3, 1), lambda i: (i, 0)),
            pl.BlockSpec((1, _D), lambda i: (0, 0)),
            pl.BlockSpec((1, 1), lambda i: (0, 0)),
        ],
        out_specs=pl.BlockSpec((1, _G), lambda i: (0, 0)),
        out_shape=jax.ShapeDtypeStruct((1, _G), jnp.float32),
        scratch_shapes=[
            pltpu.VMEM((_G, _D), jnp.float32),
            pltpu.VMEM((_G, _D), jnp.float32),
            pltpu.VMEM((_G, _D), jnp.float32),
        ],
    )(acc, b2r, rmat, bat2d, wlint, blin2d)


def _splat(vec, k):
    dnums = lax.GatherDimensionNumbers(
        offset_dims=(), collapsed_slice_dims=(0,), start_index_map=(0,))
    idx = jnp.full((16, 1), k, jnp.int32)
    return lax.gather(vec, idx, dnums, (1,),
                      mode=lax.GatherScatterMode.PROMISE_IN_BOUNDS)


_sc_mesh = plsc.VectorSubcoreMesh(
    core_axis_name="c", subcore_axis_name="s", num_cores=2, num_subcores=16)


@functools.partial(
    pl.kernel,
    out_type=jax.ShapeDtypeStruct((2, _NA, _DW), jnp.float32),
    mesh=_sc_mesh,
    scratch_types=[
        pltpu.VMEM_SHARED((_NA, _DW), jnp.float32),
        pltpu.VMEM((_SS, _K), jnp.int32),
        pltpu.VMEM((_SS, _K), jnp.int32),
        pltpu.VMEM((_SS, _K), jnp.int32),
        pltpu.VMEM((_SS, _K), jnp.int32),
        pltpu.VMEM((_K, _DW), jnp.float32),
        pltpu.VMEM((_K, _DW), jnp.float32),
        pltpu.VMEM((_K, 16), jnp.float32),
        pltpu.VMEM((_K, 16), jnp.float32),
        pltpu.SemaphoreType.DMA,
        pltpu.SemaphoreType.DMA,
        pltpu.SemaphoreType.DMA,
        pltpu.SemaphoreType.DMA,
        pltpu.SemaphoreType.DMA,
        pltpu.SemaphoreType.DMA,
    ],
    compiler_params=pltpu.CompilerParams(
        needs_layout_passes=False, use_tc_tiling_on_sc=False),
)
def _sc_layer(th, tad, srcp, dstp, acc_out,
              acc, sia0, sia1, dia0, dia1, hbuf0, hbuf1, adbuf0, adbuf1,
              gh0, gh1, ga0, ga1, sc0, sc1):
    cid = lax.axis_index("c")
    sid = lax.axis_index("s")
    wid = sid * 2 + cid
    zed = jnp.zeros((16,), jnp.float32)
    sias = (sia0, sia1)
    dias = (dia0, dia1)
    hbufs = (hbuf0, hbuf1)
    adbufs = (adbuf0, adbuf1)
    ghs = (gh0, gh1)
    gas = (ga0, ga1)
    scs = (sc0, sc1)

    def _zrow(r, carry):
        for j in range(_DW // 16):
            hbuf0[r, pl.ds(j * 16, 16)] = zed
        return carry

    lax.fori_loop(0, _K, _zrow, 0)
    rows_per_sub = _NA // 16
    nfull = rows_per_sub // _K
    rem = rows_per_sub % _K
    for z in range(nfull):
        base = sid * rows_per_sub + z * _K
        pltpu.sync_copy(hbuf0, acc.at[pl.ds(base, _K), :])
    if rem:
        pltpu.sync_copy(hbuf0.at[pl.ds(0, rem), :],
                        acc.at[pl.ds(sid * rows_per_sub + nfull * _K, rem), :])
    plsc.subcore_barrier()

    lanes = lax.iota(jnp.int32, 16)
    sub = lanes >> 2
    headv = lanes & 3

    def _idx_fetch(s, ib):
        row0 = wid * _CHUNKS + s * _SS
        pltpu.sync_copy(srcp.at[pl.ds(row0, _SS), :], sias[ib])
        pltpu.sync_copy(dstp.at[pl.ds(row0, _SS), :], dias[ib])

    def _gath(ib, k, hb):
        pltpu.async_copy(th.at[sias[ib].at[k]], hbufs[hb], ghs[hb])
        pltpu.async_copy(tad.at[dias[ib].at[k]], adbufs[hb], gas[hb])

    def _compute(b):
        hb = hbufs[b]
        ab = adbufs[b]

        def _groups(gi, carry2):
            for gu in range(_GU):
                g = gi * _GU + gu
                rowv = sub + 4 * g
                asv = plsc.load_gather(hb, [rowv, _D + headv])
                adv = plsc.load_gather(ab, [rowv, headv])
                e = asv + adv
                e = jnp.where(e > 0, e, 0.2 * e)
                ex = jnp.exp(e)
                plsc.store_scatter(hb, [rowv, _D + headv], ex)
                for j in range(4):
                    r = 4 * g + j
                    for hh in range(_H):
                        spl = _splat(ex, 4 * j + hh)
                        for q in range(2):
                            c0 = hh * 32 + q * 16
                            hb[r, pl.ds(c0, 16)] = hb[r, pl.ds(c0, 16)] * spl
            return carry2

        lax.fori_loop(0, (_K // 4) // _GU, _groups, 0)

    _idx_fetch(0, 0)
    _gath(0, 0, 0)

    def _pair(i2, carry):
        for si in range(2):
            for k in range(_SS):
                s = i2 * 2 + si
                ci = s * _SS + k
                b = (si * _SS + k) & 1
                nb = 1 - b

                if k == 0:
                    @pl.when(s + 1 < _NS)
                    def _pfidx():
                        @pl.when(ci >= 1)
                        def _drain0():
                            pltpu.make_async_copy(
                                hbufs[nb], acc.at[dias[si].at[0]],
                                scs[nb]).wait()
                        _idx_fetch(s + 1, 1 - si)

                @pl.when(ci + 1 < _CHUNKS)
                def _pref():
                    if k == 0:
                        cond = (ci >= 1) & (s + 1 >= _NS)
                    else:
                        cond = ci >= 1

                    @pl.when(cond)
                    def _drain():
                        pltpu.make_async_copy(
                            hbufs[nb], acc.at[dias[si].at[0]], scs[nb]).wait()
                    nk = k + 1
                    if nk < _SS:
                        _gath(si, nk, nb)
                    else:
                        _gath(1 - si, 0, nb)

                pltpu.make_async_copy(th.at[sias[si].at[k]],
                                      hbufs[b], ghs[b]).wait()
                pltpu.make_async_copy(tad.at[dias[si].at[k]],
                                      adbufs[b], gas[b]).wait()
                _compute(b)
                pltpu.async_copy(hbufs[b], acc.at[dias[si].at[k]],
                                 scs[b], add=True)
        return carry

    lax.fori_loop(0, _NS // 2, _pair, 0)
    pltpu.make_async_copy(hbufs[0], acc.at[dias[1].at[0]], scs[0]).wait()
    pltpu.make_async_copy(hbufs[1], acc.at[dias[1].at[0]], scs[1]).wait()
    plsc.subcore_barrier()
    base = sid * rows_per_sub
    pltpu.sync_copy(acc.at[pl.ds(base, rows_per_sub), :],
                    acc_out.at[cid, pl.ds(base, rows_per_sub), :])


def _att_mat(att):
    cols = jnp.repeat(jnp.arange(_H), 32)
    m = jnp.zeros((_D, 16), jnp.float32)
    return m.at[jnp.arange(_D), cols].set(att.reshape(-1))


def kernel(x, edge_index, batch, W1, att_src1, att_dst1, b1,
           W2, att_src2, att_dst2, b2, W_lin, b_lin):
    f32 = jnp.float32
    x_pad = jnp.zeros((_NA, 160), f32).at[:_N, :159].set(x.astype(f32))
    w1p = jnp.zeros((160, _D), f32).at[:159, :].set(W1.astype(f32))
    as1m = _att_mat(att_src1)
    ad1m = _att_mat(att_dst1)
    as2m = _att_mat(att_src2)
    ad2m = _att_mat(att_dst2)
    rmat = (jnp.arange(16)[:, None] == (jnp.arange(_D) // 32)[None, :]).astype(f32)
    b1r = b1.reshape(1, _D).astype(f32)
    b2r = b2.reshape(1, _D).astype(f32)
    wlint = W_lin.reshape(1, _D).astype(f32)
    blin2d = b_lin.reshape(1, 1).astype(f32)
    src = edge_index[0].astype(jnp.int32)
    dst = edge_index[1].astype(jnp.int32)
    srcp = jnp.zeros((_EP,), jnp.int32).at[:_E].set(src).reshape(_EP // _K, _K)
    dstp = jnp.full((_EP,), _N, jnp.int32).at[:_E].set(dst).reshape(_EP // _K, _K)
    bat2d = jnp.full((_NA, 1), _G, jnp.int32).at[:_N, 0].set(batch.astype(jnp.int32))

    th1, tad1 = _tc1(x_pad, w1p, as1m, ad1m)
    acc1 = _sc_layer(th1, tad1, srcp, dstp)
    th2, tad2 = _tc2(acc1, b1r, W2.astype(f32), as2m, ad2m, rmat)
    acc2 = _sc_layer(th2, tad2, srcp, dstp)
    return _tc3(acc2, b2r, rmat, bat2d, wlint, blin2d)
